# Initial kernel scaffold; baseline (speedup 1.0000x reference)
import jax
import jax.numpy as jnp
from jax.experimental import pallas as pl


def kernel(features, feature_values, emb_table, bias_table, W1, b1, W2, b2, Wp, bp):
    raise NotImplementedError("write your pallas kernel here")


# speedup vs baseline: 1.2826x; 1.2826x over previous
import functools

import jax
import jax.numpy as jnp
from jax import lax
from jax.experimental import pallas as pl
from jax.experimental.pallas import tpu as pltpu
from jax.experimental.pallas import tpu_sc as plsc

FEATURE_SIZE = 1000000
F = 26
D = 16
B = 16384
IN_DIM = F * D
R = B * F

NC = 2
NS = 16
NW = NC * NS
PER_W = R // NW
GSZ = 128
KG = 8
STEP = KG * GSZ
NSTEPS = PER_W // STEP


def _sc_gather(feat2, emb_table, bias_table):
    mesh = plsc.VectorSubcoreMesh(core_axis_name="c", subcore_axis_name="s")

    @functools.partial(
        pl.kernel,
        mesh=mesh,
        compiler_params=pltpu.CompilerParams(use_tc_tiling_on_sc=False),
        out_type=[
            jax.ShapeDtypeStruct((R, D), jnp.float32),
            jax.ShapeDtypeStruct((R,), jnp.float32),
        ],
        scratch_types=[
            pltpu.VMEM((KG, GSZ), jnp.int32),
            pltpu.VMEM((STEP, D), jnp.float32),
            pltpu.VMEM((STEP,), jnp.float32),
            pltpu.SemaphoreType.DMA,
            pltpu.SemaphoreType.DMA,
        ],
    )
    def k(feat_hbm, emb_hbm, bias_hbm, rows_out, bias_out, idx_v, rows_v, brows_v, sem_e, sem_b):
        wid = lax.axis_index("s") * NC + lax.axis_index("c")
        row_base = wid * PER_W
        blk_base = wid * (PER_W // GSZ)

        def body(g, carry):
            off = row_base + g * STEP
            boff = blk_base + g * KG
            pltpu.sync_copy(feat_hbm.at[pl.ds(boff, KG)], idx_v)
            waits = []
            for j in range(KG):
                waits.append(pltpu.async_copy(
                    emb_hbm.at[idx_v.at[j]],
                    rows_v.at[pl.ds(j * GSZ, GSZ)], sem_e))
                waits.append(pltpu.async_copy(
                    bias_hbm.at[idx_v.at[j]],
                    brows_v.at[pl.ds(j * GSZ, GSZ)], sem_b))
            for w in waits:
                w.wait()
            pltpu.sync_copy(rows_v, rows_out.at[pl.ds(off, STEP)])
            pltpu.sync_copy(brows_v, bias_out.at[pl.ds(off, STEP)])
            return carry

        lax.fori_loop(0, NSTEPS, body, 0)

    return k(feat2, emb_table, bias_table)


def _tc_body(raw_ref, fv_ref, bg_ref, e_ref, s_ref, w1_ref, b1_ref, w2_ref,
             b2_ref, wp2_ref, wp0_ref, wp1_ref, bp_ref, out_ref):
    fv = fv_ref[...]
    fv_e = jnp.dot(fv, e_ref[...], preferred_element_type=jnp.float32)
    scaled = raw_ref[...] * fv_e
    s = jnp.dot(scaled, s_ref[...], preferred_element_type=jnp.float32)
    second = 0.5 * (jnp.sum(s * s, axis=1, keepdims=True)
                    - jnp.sum(scaled * scaled, axis=1, keepdims=True))
    first = jnp.sum(bg_ref[...] * fv, axis=1, keepdims=True)
    h = jnp.maximum(jnp.dot(scaled, w1_ref[...],
                            preferred_element_type=jnp.float32) + b1_ref[...], 0.0)
    y = jnp.maximum(jnp.dot(h, w2_ref[...],
                            preferred_element_type=jnp.float32) + b2_ref[...], 0.0)
    o = (jnp.dot(y, wp2_ref[...], preferred_element_type=jnp.float32)
         + first * wp0_ref[0, 0] + second * wp1_ref[0, 0] + bp_ref[0, 0])
    out_ref[...] = o


def _tc_compute(raw, fv, biasg, E, S, W1, b1, W2, b2, Wp2, wp0, wp1, bp):
    bm = 512
    grid = (B // bm,)
    fixed = lambda i: (0, 0)
    return pl.pallas_call(
        _tc_body,
        grid=grid,
        in_specs=[
            pl.BlockSpec((bm, IN_DIM), lambda i: (i, 0)),
            pl.BlockSpec((bm, F), lambda i: (i, 0)),
            pl.BlockSpec((bm, F), lambda i: (i, 0)),
            pl.BlockSpec((F, IN_DIM), fixed),
            pl.BlockSpec((IN_DIM, D), fixed),
            pl.BlockSpec((IN_DIM, IN_DIM), fixed),
            pl.BlockSpec((1, IN_DIM), fixed),
            pl.BlockSpec((IN_DIM, IN_DIM), fixed),
            pl.BlockSpec((1, IN_DIM), fixed),
            pl.BlockSpec((IN_DIM, 1), fixed),
            pl.BlockSpec((1, 1), fixed),
            pl.BlockSpec((1, 1), fixed),
            pl.BlockSpec((1, 1), fixed),
        ],
        out_specs=pl.BlockSpec((bm, 1), lambda i: (i, 0)),
        out_shape=jax.ShapeDtypeStruct((B, 1), jnp.float32),
    )(raw, fv, biasg, E, S, W1, b1, W2, b2, Wp2, wp0, wp1, bp)


def kernel(features, feature_values, emb_table, bias_table, W1, b1, W2, b2, Wp, bp):
    feat2 = features.reshape(R // GSZ, GSZ)
    raw, biasg = _sc_gather(feat2, emb_table, bias_table.reshape(-1))
    raw = raw.reshape(B, IN_DIM)
    biasg = biasg.reshape(B, F)
    E = jnp.kron(jnp.eye(F, dtype=jnp.float32), jnp.ones((1, D), jnp.float32))
    S = jnp.tile(jnp.eye(D, dtype=jnp.float32), (F, 1))
    out = _tc_compute(raw, feature_values, biasg, E, S, W1,
                      b1.reshape(1, IN_DIM), W2, b2.reshape(1, IN_DIM),
                      Wp[2:], Wp[0:1], Wp[1:2], bp.reshape(1, 1))
    return out.reshape(-1)
